# baseline (speedup 1.0000x reference)
import functools

import jax
import jax.numpy as jnp
from jax import lax
from jax.experimental import pallas as pl
from jax.experimental.pallas import tpu as pltpu
from jax.experimental.pallas import tpu_sc as plsc

HIDDEN = 768
FFN = 3072
E = 8
N_TOK = 1024
NA = 2 * N_TOK
TBR = 128
NS = 24
NW = 32
GPT = (NS * TBR) // NW
TPT = N_TOK // NW


def _router_body(x_ref, wg_ref, ei_ref, ew_ref):
    x = x_ref[...]
    logits = lax.dot_general(x, wg_ref[...], (((1,), (1,)), ((), ())),
                             preferred_element_type=jnp.float32)
    m = jax.nn.softmax(logits, axis=-1)
    i1 = jnp.argmax(m, axis=-1).astype(jnp.int32)
    w1 = jnp.max(m, axis=-1)
    col = lax.broadcasted_iota(jnp.int32, m.shape, 1)
    m2 = jnp.where(col == i1[:, None], -jnp.inf, m)
    i2 = jnp.argmax(m2, axis=-1).astype(jnp.int32)
    w2 = jnp.max(m2, axis=-1)
    d = w1 + w2
    ei_ref[0, :] = i1
    ei_ref[1, :] = i2
    ew_ref[0, :] = w1 / d
    ew_ref[1, :] = w2 / d


def _router(x, Wg):
    return pl.pallas_call(
        _router_body,
        out_shape=(
            jax.ShapeDtypeStruct((2, N_TOK), jnp.int32),
            jax.ShapeDtypeStruct((2, N_TOK), jnp.float32),
        ),
    )(x, Wg)


def _sort_body(eid_hbm, stok_hbm, pos_hbm, sexp_hbm, nb_hbm,
               eid_v, stok_v, sa_v, pos_v, sexp_v, nb_v):
    cid = lax.axis_index("c")
    sid = lax.axis_index("s")

    @pl.when(jnp.logical_and(cid == 0, sid == 0))
    def _():
        pltpu.sync_copy(eid_hbm, eid_v)
        ioto = lax.iota(jnp.int32, 16)
        zeros = jnp.zeros((16,), jnp.int32)

        def initloop(i, _):
            stok_v[pl.ds(i * 16, 16)] = zeros
            sa_v[pl.ds(i * 16, 16)] = zeros + NA
            return 0
        lax.fori_loop(0, (NS * TBR + 16) // 16, initloop, 0)

        def histloop(i, cnts):
            v = eid_v[pl.ds(i * 16, 16)]
            return tuple(
                cnts[e] + plsc.all_reduce_population_count(v == e)
                for e in range(E))
        cnts = lax.fori_loop(
            0, NA // 16, histloop,
            tuple(jnp.zeros((16,), jnp.int32) for _ in range(E)))
        counts = [cnts[e][0] for e in range(E)]
        nbs = [(counts[e] + (TBR - 1)) >> 7 for e in range(E)]
        starts = []
        acc = jnp.int32(0)
        for e in range(E):
            starts.append(acc)
            acc = acc + nbs[e]
        num_blocks = acc
        ends = [starts[e] + nbs[e] for e in range(E)]

        for r in range(2):
            sl = ioto + r * 16
            ecnt = zeros
            for e in range(E):
                ecnt = ecnt + jnp.where(ends[e] <= sl, 1, 0)
            sexp_v[pl.ds(r * 16, 16)] = jnp.minimum(ecnt, E - 1)
        nb_v[...] = zeros + num_blocks

        for e in range(E):
            def p2loop(i, c, e=e):
                v = eid_v[pl.ds(i * 16, 16)]
                a_vec = ioto + i * 16
                m = v == e
                plsc.store_compressed(stok_v.at[pl.ds(c, 16)],
                                      a_vec & (N_TOK - 1), mask=m)
                plsc.store_compressed(sa_v.at[pl.ds(c, 16)], a_vec, mask=m)
                return c + plsc.all_reduce_population_count(m)[0]
            lax.fori_loop(0, NA // 16, p2loop, starts[e] * TBR)

        def invloop(j, _):
            av = sa_v[pl.ds(j * 16, 16)]
            plsc.store_scatter(pos_v, [av], ioto + j * 16)
            return 0
        lax.fori_loop(0, (NS * TBR) // 16, invloop, 0)

        pltpu.sync_copy(stok_v.at[pl.ds(0, NS * TBR)], stok_hbm)
        pltpu.sync_copy(pos_v.at[pl.ds(0, NA)], pos_hbm)
        pltpu.sync_copy(sexp_v, sexp_hbm)
        pltpu.sync_copy(nb_v, nb_hbm)


def _sort(eid):
    return pl.kernel(
        _sort_body,
        out_type=(
            jax.ShapeDtypeStruct((NS * TBR,), jnp.int32),
            jax.ShapeDtypeStruct((NA,), jnp.int32),
            jax.ShapeDtypeStruct((32,), jnp.int32),
            jax.ShapeDtypeStruct((16,), jnp.int32),
        ),
        mesh=plsc.VectorSubcoreMesh(core_axis_name="c", subcore_axis_name="s"),
        compiler_params=pltpu.CompilerParams(needs_layout_passes=False),
        scratch_types=[
            pltpu.VMEM((NA,), jnp.int32),
            pltpu.VMEM((NS * TBR + 16,), jnp.int32),
            pltpu.VMEM((NS * TBR + 16,), jnp.int32),
            pltpu.VMEM((NA + 16,), jnp.int32),
            pltpu.VMEM((32,), jnp.int32),
            pltpu.VMEM((16,), jnp.int32),
        ],
    )(eid)


def _gather_body(stok_hbm, nb_hbm, x_hbm, xs_hbm, idx_v, rows_v, nb_v, sem):
    wid = lax.axis_index("s") * 2 + lax.axis_index("c")
    base = wid * GPT
    pltpu.sync_copy(nb_hbm, nb_v)
    nrows = nb_v[pl.ds(0, 16)][0] * TBR

    @pl.when(base < nrows)
    def _():
        pltpu.sync_copy(stok_hbm.at[pl.ds(base, GPT)], idx_v)
        copies = [
            pltpu.async_copy(x_hbm.at[idx_v.at[pl.ds(k * 8, 8)]],
                             rows_v.at[pl.ds(k * 8, 8)], sem)
            for k in range(GPT // 8)
        ]
        for cp in copies:
            cp.wait()
        pltpu.sync_copy(rows_v, xs_hbm.at[pl.ds(base, GPT)])


def _gather(stok, nbv, x):
    return pl.kernel(
        _gather_body,
        out_type=jax.ShapeDtypeStruct((NS * TBR, HIDDEN // 2), jnp.int32),
        mesh=plsc.VectorSubcoreMesh(core_axis_name="c", subcore_axis_name="s"),
        scratch_types=[
            pltpu.VMEM((GPT,), jnp.int32),
            pltpu.VMEM((GPT, HIDDEN // 2), jnp.int32),
            pltpu.VMEM((16,), jnp.int32),
            pltpu.SemaphoreType.DMA,
        ],
    )(stok, nbv, x)


def _mlp_body(sexp_ref, nb_ref, xs_ref, w1_ref, w2_ref, w3_ref, ys_ref):
    s = pl.program_id(0)

    @pl.when(s < nb_ref[0])
    def _():
        xi = xs_ref[...]
        lo = lax.bitcast_convert_type(xi << 16, jnp.float32)
        hi = lax.bitcast_convert_type(xi & jnp.int32(-65536), jnp.float32)
        x = jnp.stack([lo, hi], axis=-1).reshape(TBR, HIDDEN)
        h1 = jnp.maximum(
            lax.dot_general(x, w1_ref[0], (((1,), (1,)), ((), ())),
                            preferred_element_type=jnp.float32), 0.0)
        h2 = jnp.maximum(
            lax.dot_general(h1, w2_ref[0], (((1,), (1,)), ((), ())),
                            preferred_element_type=jnp.float32), 0.0)
        ys_ref[...] = lax.dot_general(h2, w3_ref[0], (((1,), (1,)), ((), ())),
                                      preferred_element_type=jnp.float32)


def _mlp(sexp, nbv, xs, W1, W2, W3):
    grid_spec = pltpu.PrefetchScalarGridSpec(
        num_scalar_prefetch=2,
        grid=(NS,),
        in_specs=[
            pl.BlockSpec((TBR, HIDDEN // 2), lambda s, se, nb: (s, 0)),
            pl.BlockSpec((1, HIDDEN, HIDDEN), lambda s, se, nb: (se[s], 0, 0)),
            pl.BlockSpec((1, HIDDEN, HIDDEN), lambda s, se, nb: (se[s], 0, 0)),
            pl.BlockSpec((1, FFN, HIDDEN), lambda s, se, nb: (se[s], 0, 0)),
        ],
        out_specs=pl.BlockSpec((TBR, FFN), lambda s, se, nb: (s, 0)),
    )
    return pl.pallas_call(
        _mlp_body,
        grid_spec=grid_spec,
        out_shape=jax.ShapeDtypeStruct((NS * TBR, FFN), jnp.float32),
        compiler_params=pltpu.CompilerParams(
            dimension_semantics=("arbitrary",),
            vmem_limit_bytes=100 * 1024 * 1024,
        ),
    )(sexp, nbv, xs, W1, W2, W3)


_CCH = 8
_NCH = TPT // _CCH


def _combine_body(pos_hbm, wgt_hbm, ys_hbm, out_hbm,
                  p0_v, p1_v, w0_v, w1_v,
                  rA0, rB0, rA1, rB1, acc0, sem, osem):
    wid = lax.axis_index("s") * 2 + lax.axis_index("c")
    tb = wid * TPT
    pltpu.sync_copy(pos_hbm.at[pl.ds(tb, TPT)], p0_v)
    pltpu.sync_copy(pos_hbm.at[pl.ds(N_TOK + tb, TPT)], p1_v)
    pltpu.sync_copy(wgt_hbm.at[pl.ds(tb, TPT)], w0_v)
    pltpu.sync_copy(wgt_hbm.at[pl.ds(N_TOK + tb, TPT)], w1_v)
    w0a = w0_v[pl.ds(0, 16)]
    w0b = w0_v[pl.ds(16, 16)]
    w1a = w1_v[pl.ds(0, 16)]
    w1b = w1_v[pl.ds(16, 16)]
    rows = [(rA0, rB0), (rA1, rB1)]

    def fire(c, buf):
        A, B = rows[buf]
        ca = pltpu.async_copy(ys_hbm.at[p0_v.at[pl.ds(c * _CCH, _CCH)]], A, sem)
        cb = pltpu.async_copy(ys_hbm.at[p1_v.at[pl.ds(c * _CCH, _CCH)]], B, sem)
        return ca, cb

    pend = fire(0, 0)
    ocopy = None
    for c in range(_NCH):
        nxt = fire(c + 1, (c + 1) % 2) if c + 1 < _NCH else None
        pend[0].wait()
        pend[1].wait()
        A, B = rows[c % 2]
        acc = acc0
        if ocopy is not None:
            ocopy.wait()
        wa = [(w0a if c * _CCH + j < 16 else w0b)[(c * _CCH + j) % 16]
              for j in range(_CCH)]
        wb = [(w1a if c * _CCH + j < 16 else w1b)[(c * _CCH + j) % 16]
              for j in range(_CCH)]

        def addloop(r, _, A=A, B=B, acc=acc, wa=wa, wb=wb):
            for j in range(_CCH):
                acc[j, pl.ds(r * 16, 16)] = (
                    A[j, pl.ds(r * 16, 16)] * wa[j]
                    + B[j, pl.ds(r * 16, 16)] * wb[j])
            return 0
        lax.fori_loop(0, FFN // 16, addloop, 0, unroll=4)
        ocopy = pltpu.async_copy(
            acc, out_hbm.at[pl.ds(tb + c * _CCH, _CCH)], osem)
        pend = nxt
    ocopy.wait()


def _combine(pos, wgt, ys):
    return pl.kernel(
        _combine_body,
        out_type=jax.ShapeDtypeStruct((N_TOK, FFN), jnp.float32),
        mesh=plsc.VectorSubcoreMesh(core_axis_name="c", subcore_axis_name="s"),
        scratch_types=[
            pltpu.VMEM((TPT,), jnp.int32),
            pltpu.VMEM((TPT,), jnp.int32),
            pltpu.VMEM((TPT,), jnp.float32),
            pltpu.VMEM((TPT,), jnp.float32),
            pltpu.VMEM((_CCH, FFN), jnp.float32),
            pltpu.VMEM((_CCH, FFN), jnp.float32),
            pltpu.VMEM((_CCH, FFN), jnp.float32),
            pltpu.VMEM((_CCH, FFN), jnp.float32),
            pltpu.VMEM((_CCH, FFN), jnp.float32),
            pltpu.SemaphoreType.DMA,
            pltpu.SemaphoreType.DMA,
        ],
    )(pos, wgt, ys)


def kernel(hidden_states, Wg, W1, W2, W3):
    b, ch, h, w = hidden_states.shape
    x = jnp.transpose(hidden_states, (0, 2, 3, 1)).reshape(-1, ch)
    ei, ew = _router(x, Wg)
    eid = ei.reshape(NA)
    wgt = ew.reshape(NA)
    stok, pos, sexp, nbv = _sort(eid)
    xi = lax.bitcast_convert_type(
        hidden_states.astype(jnp.bfloat16)
        .transpose(0, 2, 3, 1).reshape(N_TOK, HIDDEN // 2, 2),
        jnp.int32)
    xs = _gather(stok, nbv, xi)
    ys = _mlp(sexp, nbv, xs, W1, W2, W3)
    out_flat = _combine(pos, wgt, ys)
    out = out_flat.reshape(b, h, w, FFN)
    return jnp.transpose(out, (0, 3, 1, 2))


# speedup vs baseline: 2.8369x; 2.8369x over previous
import functools

import jax
import jax.numpy as jnp
from jax import lax
from jax.experimental import pallas as pl
from jax.experimental.pallas import tpu as pltpu
from jax.experimental.pallas import tpu_sc as plsc

HIDDEN = 768
FFN = 3072
E = 8
N_TOK = 1024
NA = 2 * N_TOK
TBR = 128
NS = 24
NW = 32
GPT = (NS * TBR) // NW
TPT = N_TOK // NW


def _router_body(x_ref, wg_ref, ei_ref, ew_ref):
    x = x_ref[...]
    logits = lax.dot_general(x, wg_ref[...], (((1,), (1,)), ((), ())),
                             preferred_element_type=jnp.float32)
    m = jax.nn.softmax(logits, axis=-1)
    i1 = jnp.argmax(m, axis=-1).astype(jnp.int32)
    w1 = jnp.max(m, axis=-1)
    col = lax.broadcasted_iota(jnp.int32, m.shape, 1)
    m2 = jnp.where(col == i1[:, None], -jnp.inf, m)
    i2 = jnp.argmax(m2, axis=-1).astype(jnp.int32)
    w2 = jnp.max(m2, axis=-1)
    d = w1 + w2
    ei_ref[0, :] = i1
    ei_ref[1, :] = i2
    ew_ref[0, :] = w1 / d
    ew_ref[1, :] = w2 / d


def _router(x, Wg):
    return pl.pallas_call(
        _router_body,
        out_shape=(
            jax.ShapeDtypeStruct((2, N_TOK), jnp.int32),
            jax.ShapeDtypeStruct((2, N_TOK), jnp.float32),
        ),
    )(x, Wg)


def _sort_body(eid_hbm, stok_hbm, pos_hbm, sexp_hbm, nb_hbm,
               eid_v, stok_v, sa_v, pos_v, sexp_v, nb_v):
    cid = lax.axis_index("c")
    sid = lax.axis_index("s")

    @pl.when(jnp.logical_and(cid == 0, sid == 0))
    def _():
        pltpu.sync_copy(eid_hbm, eid_v)
        ioto = lax.iota(jnp.int32, 16)
        zeros = jnp.zeros((16,), jnp.int32)

        def initloop(i, _):
            stok_v[pl.ds(i * 16, 16)] = zeros
            sa_v[pl.ds(i * 16, 16)] = zeros + NA
            return 0
        lax.fori_loop(0, (NS * TBR + 16) // 16, initloop, 0)

        def histloop(i, cnts):
            v = eid_v[pl.ds(i * 16, 16)]
            return tuple(
                cnts[e] + plsc.all_reduce_population_count(v == e)
                for e in range(E))
        cnts = lax.fori_loop(
            0, NA // 16, histloop,
            tuple(jnp.zeros((16,), jnp.int32) for _ in range(E)))
        counts = [cnts[e][0] for e in range(E)]
        nbs = [(counts[e] + (TBR - 1)) >> 7 for e in range(E)]
        starts = []
        acc = jnp.int32(0)
        for e in range(E):
            starts.append(acc)
            acc = acc + nbs[e]
        num_blocks = acc
        ends = [starts[e] + nbs[e] for e in range(E)]

        for r in range(2):
            sl = ioto + r * 16
            ecnt = zeros
            for e in range(E):
                ecnt = ecnt + jnp.where(ends[e] <= sl, 1, 0)
            sexp_v[pl.ds(r * 16, 16)] = jnp.minimum(ecnt, E - 1)
        nb_v[...] = zeros + num_blocks

        for e in range(E):
            def p2loop(i, c, e=e):
                v = eid_v[pl.ds(i * 16, 16)]
                a_vec = ioto + i * 16
                m = v == e
                plsc.store_compressed(stok_v.at[pl.ds(c, 16)],
                                      a_vec & (N_TOK - 1), mask=m)
                plsc.store_compressed(sa_v.at[pl.ds(c, 16)], a_vec, mask=m)
                return c + plsc.all_reduce_population_count(m)[0]
            lax.fori_loop(0, NA // 16, p2loop, starts[e] * TBR)

        def invloop(j, _):
            av = sa_v[pl.ds(j * 16, 16)]
            plsc.store_scatter(pos_v, [av], ioto + j * 16)
            return 0
        lax.fori_loop(0, (NS * TBR) // 16, invloop, 0)

        pltpu.sync_copy(stok_v.at[pl.ds(0, NS * TBR)], stok_hbm)
        pltpu.sync_copy(pos_v.at[pl.ds(0, NA)], pos_hbm)
        pltpu.sync_copy(sexp_v, sexp_hbm)
        pltpu.sync_copy(nb_v, nb_hbm)


def _sort(eid):
    return pl.kernel(
        _sort_body,
        out_type=(
            jax.ShapeDtypeStruct((NS * TBR,), jnp.int32),
            jax.ShapeDtypeStruct((NA,), jnp.int32),
            jax.ShapeDtypeStruct((32,), jnp.int32),
            jax.ShapeDtypeStruct((16,), jnp.int32),
        ),
        mesh=plsc.VectorSubcoreMesh(core_axis_name="c", subcore_axis_name="s"),
        compiler_params=pltpu.CompilerParams(needs_layout_passes=False),
        scratch_types=[
            pltpu.VMEM((NA,), jnp.int32),
            pltpu.VMEM((NS * TBR + 16,), jnp.int32),
            pltpu.VMEM((NS * TBR + 16,), jnp.int32),
            pltpu.VMEM((NA + 16,), jnp.int32),
            pltpu.VMEM((32,), jnp.int32),
            pltpu.VMEM((16,), jnp.int32),
        ],
    )(eid)


def _gather_body(stok_hbm, nb_hbm, x_hbm, xs_hbm, idx_v, rows_v, nb_v, sem):
    wid = lax.axis_index("s") * 2 + lax.axis_index("c")
    base = wid * GPT
    pltpu.sync_copy(nb_hbm, nb_v)
    nrows = nb_v[pl.ds(0, 16)][0] * TBR

    @pl.when(base < nrows)
    def _():
        pltpu.sync_copy(stok_hbm.at[pl.ds(base, GPT)], idx_v)
        copies = [
            pltpu.async_copy(x_hbm.at[idx_v.at[pl.ds(k * 8, 8)]],
                             rows_v.at[pl.ds(k * 8, 8)], sem)
            for k in range(GPT // 8)
        ]
        for cp in copies:
            cp.wait()
        pltpu.sync_copy(rows_v, xs_hbm.at[pl.ds(base, GPT)])


def _gather(stok, nbv, x):
    return pl.kernel(
        _gather_body,
        out_type=jax.ShapeDtypeStruct((NS * TBR, HIDDEN // 2), jnp.int32),
        mesh=plsc.VectorSubcoreMesh(core_axis_name="c", subcore_axis_name="s"),
        scratch_types=[
            pltpu.VMEM((GPT,), jnp.int32),
            pltpu.VMEM((GPT, HIDDEN // 2), jnp.int32),
            pltpu.VMEM((16,), jnp.int32),
            pltpu.SemaphoreType.DMA,
        ],
    )(stok, nbv, x)


def _mlp_body(sexp_ref, nb_ref, xs_ref, w1_ref, w2_ref, w3_ref, ys_ref):
    s = pl.program_id(0)

    @pl.when(s < nb_ref[0])
    def _():
        xi = xs_ref[...]
        lo = lax.bitcast_convert_type(xi << 16, jnp.float32)
        hi = lax.bitcast_convert_type(xi & jnp.int32(-65536), jnp.float32)
        x = jnp.concatenate([lo, hi], axis=1)
        h1 = jnp.maximum(
            lax.dot_general(x, w1_ref[0], (((1,), (1,)), ((), ())),
                            preferred_element_type=jnp.float32), 0.0)
        h2 = jnp.maximum(
            lax.dot_general(h1, w2_ref[0], (((1,), (1,)), ((), ())),
                            preferred_element_type=jnp.float32), 0.0)
        ys_ref[...] = lax.dot_general(h2, w3_ref[0], (((1,), (1,)), ((), ())),
                                      preferred_element_type=jnp.float32)


def _mlp(sexp, nbv, xs, W1, W2, W3):
    grid_spec = pltpu.PrefetchScalarGridSpec(
        num_scalar_prefetch=2,
        grid=(NS,),
        in_specs=[
            pl.BlockSpec((TBR, HIDDEN // 2), lambda s, se, nb: (s, 0)),
            pl.BlockSpec((1, HIDDEN, HIDDEN), lambda s, se, nb: (se[s], 0, 0)),
            pl.BlockSpec((1, HIDDEN, HIDDEN), lambda s, se, nb: (se[s], 0, 0)),
            pl.BlockSpec((1, FFN, HIDDEN), lambda s, se, nb: (se[s], 0, 0)),
        ],
        out_specs=pl.BlockSpec((TBR, FFN), lambda s, se, nb: (s, 0)),
    )
    return pl.pallas_call(
        _mlp_body,
        grid_spec=grid_spec,
        out_shape=jax.ShapeDtypeStruct((NS * TBR, FFN), jnp.float32),
        compiler_params=pltpu.CompilerParams(
            dimension_semantics=("arbitrary",),
            vmem_limit_bytes=100 * 1024 * 1024,
        ),
    )(sexp, nbv, xs, W1, W2, W3)


_CCH = 8
_NCH = TPT // _CCH


def _combine_body(pos_hbm, wgt_hbm, ys_hbm, out_hbm,
                  p0_v, p1_v, w0_v, w1_v,
                  rA0, rB0, rA1, rB1, acc0, sem, osem):
    wid = lax.axis_index("s") * 2 + lax.axis_index("c")
    tb = wid * TPT
    pltpu.sync_copy(pos_hbm.at[pl.ds(tb, TPT)], p0_v)
    pltpu.sync_copy(pos_hbm.at[pl.ds(N_TOK + tb, TPT)], p1_v)
    pltpu.sync_copy(wgt_hbm.at[pl.ds(tb, TPT)], w0_v)
    pltpu.sync_copy(wgt_hbm.at[pl.ds(N_TOK + tb, TPT)], w1_v)
    w0a = w0_v[pl.ds(0, 16)]
    w0b = w0_v[pl.ds(16, 16)]
    w1a = w1_v[pl.ds(0, 16)]
    w1b = w1_v[pl.ds(16, 16)]
    rows = [(rA0, rB0), (rA1, rB1)]

    def fire(c, buf):
        A, B = rows[buf]
        ca = pltpu.async_copy(ys_hbm.at[p0_v.at[pl.ds(c * _CCH, _CCH)]], A, sem)
        cb = pltpu.async_copy(ys_hbm.at[p1_v.at[pl.ds(c * _CCH, _CCH)]], B, sem)
        return ca, cb

    pend = fire(0, 0)
    ocopy = None
    for c in range(_NCH):
        nxt = fire(c + 1, (c + 1) % 2) if c + 1 < _NCH else None
        pend[0].wait()
        pend[1].wait()
        A, B = rows[c % 2]
        acc = acc0
        if ocopy is not None:
            ocopy.wait()
        wa = [(w0a if c * _CCH + j < 16 else w0b)[(c * _CCH + j) % 16]
              for j in range(_CCH)]
        wb = [(w1a if c * _CCH + j < 16 else w1b)[(c * _CCH + j) % 16]
              for j in range(_CCH)]

        def addloop(r, _, A=A, B=B, acc=acc, wa=wa, wb=wb):
            for j in range(_CCH):
                acc[j, pl.ds(r * 16, 16)] = (
                    A[j, pl.ds(r * 16, 16)] * wa[j]
                    + B[j, pl.ds(r * 16, 16)] * wb[j])
            return 0
        lax.fori_loop(0, FFN // 16, addloop, 0, unroll=4)
        ocopy = pltpu.async_copy(
            acc, out_hbm.at[pl.ds(tb + c * _CCH, _CCH)], osem)
        pend = nxt
    ocopy.wait()


def _combine(pos, wgt, ys):
    return pl.kernel(
        _combine_body,
        out_type=jax.ShapeDtypeStruct((N_TOK, FFN), jnp.float32),
        mesh=plsc.VectorSubcoreMesh(core_axis_name="c", subcore_axis_name="s"),
        scratch_types=[
            pltpu.VMEM((TPT,), jnp.int32),
            pltpu.VMEM((TPT,), jnp.int32),
            pltpu.VMEM((TPT,), jnp.float32),
            pltpu.VMEM((TPT,), jnp.float32),
            pltpu.VMEM((_CCH, FFN), jnp.float32),
            pltpu.VMEM((_CCH, FFN), jnp.float32),
            pltpu.VMEM((_CCH, FFN), jnp.float32),
            pltpu.VMEM((_CCH, FFN), jnp.float32),
            pltpu.VMEM((_CCH, FFN), jnp.float32),
            pltpu.SemaphoreType.DMA,
            pltpu.SemaphoreType.DMA,
        ],
    )(pos, wgt, ys)


def kernel(hidden_states, Wg, W1, W2, W3):
    b, ch, h, w = hidden_states.shape
    x = jnp.transpose(hidden_states, (0, 2, 3, 1)).reshape(-1, ch)
    ei, ew = _router(x, Wg)
    eid = ei.reshape(NA)
    wgt = ew.reshape(NA)
    stok, pos, sexp, nbv = _sort(eid)
    xb = x.astype(jnp.bfloat16)
    xi = lax.bitcast_convert_type(
        jnp.stack([xb[:, :HIDDEN // 2], xb[:, HIDDEN // 2:]], axis=-1),
        jnp.int32)
    xs = _gather(stok, nbv, xi)
    ys = _mlp(sexp, nbv, xs, W1, W2, W3)
    out_flat = _combine(pos, wgt, ys)
    out = out_flat.reshape(b, h, w, FFN)
    return jnp.transpose(out, (0, 3, 1, 2))


# speedup vs baseline: 4.6881x; 1.6526x over previous
import functools

import jax
import jax.numpy as jnp
from jax import lax
from jax.experimental import pallas as pl
from jax.experimental.pallas import tpu as pltpu
from jax.experimental.pallas import tpu_sc as plsc

HIDDEN = 768
FFN = 3072
E = 8
N_TOK = 1024
NA = 2 * N_TOK
TBR = 128
NS = 24
NW = 32
GPT = (NS * TBR) // NW
TPT = N_TOK // NW


def _router_body(x_ref, wg_ref, ei_ref, ew_ref):
    x = x_ref[...]
    logits = lax.dot_general(x, wg_ref[...], (((1,), (1,)), ((), ())),
                             preferred_element_type=jnp.float32)
    m = jax.nn.softmax(logits, axis=-1)
    i1 = jnp.argmax(m, axis=-1).astype(jnp.int32)
    w1 = jnp.max(m, axis=-1)
    col = lax.broadcasted_iota(jnp.int32, m.shape, 1)
    m2 = jnp.where(col == i1[:, None], -jnp.inf, m)
    i2 = jnp.argmax(m2, axis=-1).astype(jnp.int32)
    w2 = jnp.max(m2, axis=-1)
    d = w1 + w2
    ei_ref[0, :] = i1
    ei_ref[1, :] = i2
    ew_ref[0, :] = w1 / d
    ew_ref[1, :] = w2 / d


def _router(x, Wg):
    return pl.pallas_call(
        _router_body,
        out_shape=(
            jax.ShapeDtypeStruct((2, N_TOK), jnp.int32),
            jax.ShapeDtypeStruct((2, N_TOK), jnp.float32),
        ),
    )(x, Wg)


def _sort_body(eid_hbm, stok_hbm, pos_hbm, sexp_hbm, nb_hbm,
               eid_v, stok_v, sa_v, pos_v, sexp_v, nb_v):
    cid = lax.axis_index("c")
    sid = lax.axis_index("s")

    @pl.when(jnp.logical_and(cid == 0, sid == 0))
    def _():
        pltpu.sync_copy(eid_hbm, eid_v)
        ioto = lax.iota(jnp.int32, 16)
        zeros = jnp.zeros((16,), jnp.int32)

        def initloop(i, _):
            stok_v[pl.ds(i * 16, 16)] = zeros
            sa_v[pl.ds(i * 16, 16)] = zeros + NA
            return 0
        lax.fori_loop(0, (NS * TBR + 16) // 16, initloop, 0)

        def histloop(i, cnts):
            v = eid_v[pl.ds(i * 16, 16)]
            return tuple(
                cnts[e] + plsc.all_reduce_population_count(v == e)
                for e in range(E))
        cnts = lax.fori_loop(
            0, NA // 16, histloop,
            tuple(jnp.zeros((16,), jnp.int32) for _ in range(E)))
        counts = [cnts[e][0] for e in range(E)]
        nbs = [(counts[e] + (TBR - 1)) >> 7 for e in range(E)]
        starts = []
        acc = jnp.int32(0)
        for e in range(E):
            starts.append(acc)
            acc = acc + nbs[e]
        num_blocks = acc
        ends = [starts[e] + nbs[e] for e in range(E)]

        for r in range(2):
            sl = ioto + r * 16
            ecnt = zeros
            for e in range(E):
                ecnt = ecnt + jnp.where(ends[e] <= sl, 1, 0)
            sexp_v[pl.ds(r * 16, 16)] = jnp.minimum(ecnt, E - 1)
        nb_v[...] = zeros + num_blocks

        for e in range(E):
            def p2loop(i, c, e=e):
                v = eid_v[pl.ds(i * 16, 16)]
                a_vec = ioto + i * 16
                m = v == e
                plsc.store_compressed(stok_v.at[pl.ds(c, 16)],
                                      a_vec & (N_TOK - 1), mask=m)
                plsc.store_compressed(sa_v.at[pl.ds(c, 16)], a_vec, mask=m)
                return c + plsc.all_reduce_population_count(m)[0]
            lax.fori_loop(0, NA // 16, p2loop, starts[e] * TBR)

        def invloop(j, _):
            av = sa_v[pl.ds(j * 16, 16)]
            plsc.store_scatter(pos_v, [av], ioto + j * 16)
            return 0
        lax.fori_loop(0, (NS * TBR) // 16, invloop, 0)

        pltpu.sync_copy(stok_v.at[pl.ds(0, NS * TBR)], stok_hbm)
        pltpu.sync_copy(pos_v.at[pl.ds(0, NA)], pos_hbm)
        pltpu.sync_copy(sexp_v, sexp_hbm)
        pltpu.sync_copy(nb_v, nb_hbm)


def _sort(eid):
    return pl.kernel(
        _sort_body,
        out_type=(
            jax.ShapeDtypeStruct((NS * TBR,), jnp.int32),
            jax.ShapeDtypeStruct((NA,), jnp.int32),
            jax.ShapeDtypeStruct((32,), jnp.int32),
            jax.ShapeDtypeStruct((16,), jnp.int32),
        ),
        mesh=plsc.VectorSubcoreMesh(core_axis_name="c", subcore_axis_name="s"),
        compiler_params=pltpu.CompilerParams(needs_layout_passes=False),
        scratch_types=[
            pltpu.VMEM((NA,), jnp.int32),
            pltpu.VMEM((NS * TBR + 16,), jnp.int32),
            pltpu.VMEM((NS * TBR + 16,), jnp.int32),
            pltpu.VMEM((NA + 16,), jnp.int32),
            pltpu.VMEM((32,), jnp.int32),
            pltpu.VMEM((16,), jnp.int32),
        ],
    )(eid)


def _gather_body(stok_hbm, nb_hbm, x_hbm, xs_hbm, idx_v, rows_v, nb_v, sem):
    wid = lax.axis_index("s") * 2 + lax.axis_index("c")
    base = wid * GPT
    pltpu.sync_copy(nb_hbm, nb_v)
    nrows = nb_v[pl.ds(0, 16)][0] * TBR

    @pl.when(base < nrows)
    def _():
        pltpu.sync_copy(stok_hbm.at[pl.ds(base, GPT)], idx_v)
        copies = [
            pltpu.async_copy(x_hbm.at[idx_v.at[pl.ds(k * 8, 8)]],
                             rows_v.at[pl.ds(k * 8, 8)], sem)
            for k in range(GPT // 8)
        ]
        for cp in copies:
            cp.wait()
        pltpu.sync_copy(rows_v, xs_hbm.at[pl.ds(base, GPT)])


def _gather(stok, nbv, x):
    return pl.kernel(
        _gather_body,
        out_type=jax.ShapeDtypeStruct((NS * TBR, HIDDEN // 2), jnp.int32),
        mesh=plsc.VectorSubcoreMesh(core_axis_name="c", subcore_axis_name="s"),
        scratch_types=[
            pltpu.VMEM((GPT,), jnp.int32),
            pltpu.VMEM((GPT, HIDDEN // 2), jnp.int32),
            pltpu.VMEM((16,), jnp.int32),
            pltpu.SemaphoreType.DMA,
        ],
    )(stok, nbv, x)


def _mlp_body(sexp_ref, nb_ref, xs_ref, w1_ref, w2_ref, w3_ref, ys_ref):
    s = pl.program_id(0)

    @pl.when(s < nb_ref[0])
    def _():
        xi = xs_ref[...]
        lo = lax.bitcast_convert_type(xi << 16, jnp.float32)
        hi = lax.bitcast_convert_type(xi & jnp.int32(-65536), jnp.float32)
        x = jnp.concatenate([lo, hi], axis=1)
        h1 = jnp.maximum(
            lax.dot_general(x, w1_ref[0], (((1,), (1,)), ((), ())),
                            preferred_element_type=jnp.float32), 0.0)
        h2 = jnp.maximum(
            lax.dot_general(h1, w2_ref[0], (((1,), (1,)), ((), ())),
                            preferred_element_type=jnp.float32), 0.0)
        ys_ref[...] = lax.dot_general(h2, w3_ref[0], (((1,), (1,)), ((), ())),
                                      preferred_element_type=jnp.float32)


def _mlp(sexp, nbv, xs, W1, W2, W3):
    grid_spec = pltpu.PrefetchScalarGridSpec(
        num_scalar_prefetch=2,
        grid=(NS,),
        in_specs=[
            pl.BlockSpec((TBR, HIDDEN // 2), lambda s, se, nb: (s, 0)),
            pl.BlockSpec((1, HIDDEN, HIDDEN), lambda s, se, nb: (se[s], 0, 0)),
            pl.BlockSpec((1, HIDDEN, HIDDEN), lambda s, se, nb: (se[s], 0, 0)),
            pl.BlockSpec((1, FFN, HIDDEN), lambda s, se, nb: (se[s], 0, 0)),
        ],
        out_specs=pl.BlockSpec((TBR, FFN), lambda s, se, nb: (s, 0)),
    )
    return pl.pallas_call(
        _mlp_body,
        grid_spec=grid_spec,
        out_shape=jax.ShapeDtypeStruct((NS * TBR, FFN), jnp.float32),
        compiler_params=pltpu.CompilerParams(
            dimension_semantics=("arbitrary",),
            vmem_limit_bytes=100 * 1024 * 1024,
        ),
    )(sexp, nbv, xs, W1, W2, W3)


_CCH = 8
_NCH = TPT // _CCH


def _combine_body(pos_hbm, wgt_hbm, ys_hbm, out_hbm,
                  p0_v, p1_v, w0_v, w1_v,
                  rA0, rB0, rA1, rB1, acc0, sem, osem):
    wid = lax.axis_index("s") * 2 + lax.axis_index("c")
    tb = wid * TPT
    pltpu.sync_copy(pos_hbm.at[pl.ds(tb, TPT)], p0_v)
    pltpu.sync_copy(pos_hbm.at[pl.ds(N_TOK + tb, TPT)], p1_v)
    pltpu.sync_copy(wgt_hbm.at[pl.ds(tb, TPT)], w0_v)
    pltpu.sync_copy(wgt_hbm.at[pl.ds(N_TOK + tb, TPT)], w1_v)
    w0a = w0_v[pl.ds(0, 16)]
    w0b = w0_v[pl.ds(16, 16)]
    w1a = w1_v[pl.ds(0, 16)]
    w1b = w1_v[pl.ds(16, 16)]
    rows = [(rA0, rB0), (rA1, rB1)]

    def fire(c, buf):
        A, B = rows[buf]
        ca = pltpu.async_copy(ys_hbm.at[p0_v.at[pl.ds(c * _CCH, _CCH)]], A, sem)
        cb = pltpu.async_copy(ys_hbm.at[p1_v.at[pl.ds(c * _CCH, _CCH)]], B, sem)
        return ca, cb

    pend = fire(0, 0)
    ocopy = None
    for c in range(_NCH):
        nxt = fire(c + 1, (c + 1) % 2) if c + 1 < _NCH else None
        pend[0].wait()
        pend[1].wait()
        A, B = rows[c % 2]
        acc = acc0
        if ocopy is not None:
            ocopy.wait()
        wa = [(w0a if c * _CCH + j < 16 else w0b)[(c * _CCH + j) % 16]
              for j in range(_CCH)]
        wb = [(w1a if c * _CCH + j < 16 else w1b)[(c * _CCH + j) % 16]
              for j in range(_CCH)]

        def addloop(r, _, A=A, B=B, acc=acc, wa=wa, wb=wb):
            for j in range(_CCH):
                acc[j, pl.ds(r * 16, 16)] = (
                    A[j, pl.ds(r * 16, 16)] * wa[j]
                    + B[j, pl.ds(r * 16, 16)] * wb[j])
            return 0
        lax.fori_loop(0, FFN // 16, addloop, 0, unroll=4)
        ocopy = pltpu.async_copy(
            acc, out_hbm.at[pl.ds(tb + c * _CCH, _CCH)], osem)
        pend = nxt
    ocopy.wait()


def _combine(pos, wgt, ys):
    return pl.kernel(
        _combine_body,
        out_type=jax.ShapeDtypeStruct((N_TOK, FFN), jnp.float32),
        mesh=plsc.VectorSubcoreMesh(core_axis_name="c", subcore_axis_name="s"),
        scratch_types=[
            pltpu.VMEM((TPT,), jnp.int32),
            pltpu.VMEM((TPT,), jnp.int32),
            pltpu.VMEM((TPT,), jnp.float32),
            pltpu.VMEM((TPT,), jnp.float32),
            pltpu.VMEM((_CCH, FFN), jnp.float32),
            pltpu.VMEM((_CCH, FFN), jnp.float32),
            pltpu.VMEM((_CCH, FFN), jnp.float32),
            pltpu.VMEM((_CCH, FFN), jnp.float32),
            pltpu.VMEM((_CCH, FFN), jnp.float32),
            pltpu.SemaphoreType.DMA,
            pltpu.SemaphoreType.DMA,
        ],
    )(pos, wgt, ys)


def _we_body(eid_hbm, wgt_hbm, we_hbm, eid_v, wgt_v, we_v):
    cid = lax.axis_index("c")
    sid = lax.axis_index("s")

    @pl.when(jnp.logical_and(cid == 0, sid == 0))
    def _():
        pltpu.sync_copy(eid_hbm, eid_v)
        pltpu.sync_copy(wgt_hbm, wgt_v)
        ioto = lax.iota(jnp.int32, 16)
        zf = jnp.zeros((16,), jnp.float32)

        def initloop(i, _):
            we_v[pl.ds(i * 16, 16)] = zf
            return 0
        lax.fori_loop(0, (N_TOK * E) // 16, initloop, 0)

        def scatloop(i, _):
            v = eid_v[pl.ds(i * 16, 16)]
            w = wgt_v[pl.ds(i * 16, 16)]
            a_vec = ioto + i * 16
            idx = (a_vec & (N_TOK - 1)) * E + v
            plsc.store_scatter(we_v, [idx], w)
            return 0
        lax.fori_loop(0, NA // 16, scatloop, 0)
        pltpu.sync_copy(we_v, we_hbm)


def _wescatter(eid, wgt):
    return pl.kernel(
        _we_body,
        out_type=jax.ShapeDtypeStruct((N_TOK * E,), jnp.float32),
        mesh=plsc.VectorSubcoreMesh(core_axis_name="c", subcore_axis_name="s"),
        compiler_params=pltpu.CompilerParams(needs_layout_passes=False),
        scratch_types=[
            pltpu.VMEM((NA,), jnp.int32),
            pltpu.VMEM((NA,), jnp.float32),
            pltpu.VMEM((N_TOK * E,), jnp.float32),
        ],
    )(eid, wgt)


def _dense_body(xb_ref, we_ref, w1_ref, w2_ref, w3_ref, out_ref):
    e = pl.program_id(0)
    w1 = w1_ref[0].astype(jnp.bfloat16)
    w2 = w2_ref[0].astype(jnp.bfloat16)
    w3 = w3_ref[0].astype(jnp.bfloat16)
    x = xb_ref[...]
    h1 = jnp.maximum(
        lax.dot_general(x, w1, (((1,), (1,)), ((), ())),
                        preferred_element_type=jnp.float32),
        0.0).astype(jnp.bfloat16)
    h2 = jnp.maximum(
        lax.dot_general(h1, w2, (((1,), (1,)), ((), ())),
                        preferred_element_type=jnp.float32), 0.0)
    we = we_ref[...]
    col = lax.broadcasted_iota(jnp.int32, we.shape, 1)
    wcol = jnp.sum(jnp.where(col == e, we, 0.0), axis=1, keepdims=True)
    h2w = (h2 * wcol).astype(jnp.bfloat16)
    y = lax.dot_general(h2w, w3, (((1,), (1,)), ((), ())),
                        preferred_element_type=jnp.float32)

    @pl.when(e == 0)
    def _init():
        out_ref[...] = y

    @pl.when(e != 0)
    def _acc():
        out_ref[...] += y


def _dense_mlp(xb, we, W1, W2, W3):
    return pl.pallas_call(
        _dense_body,
        grid=(E,),
        in_specs=[
            pl.BlockSpec((N_TOK, HIDDEN), lambda e: (0, 0)),
            pl.BlockSpec((N_TOK, E), lambda e: (0, 0)),
            pl.BlockSpec((1, HIDDEN, HIDDEN), lambda e: (e, 0, 0)),
            pl.BlockSpec((1, HIDDEN, HIDDEN), lambda e: (e, 0, 0)),
            pl.BlockSpec((1, FFN, HIDDEN), lambda e: (e, 0, 0)),
        ],
        out_specs=pl.BlockSpec((N_TOK, FFN), lambda e: (0, 0)),
        out_shape=jax.ShapeDtypeStruct((N_TOK, FFN), jnp.float32),
        compiler_params=pltpu.CompilerParams(
            dimension_semantics=("arbitrary",),
            vmem_limit_bytes=110 * 1024 * 1024,
        ),
    )(xb, we, W1, W2, W3)


def kernel(hidden_states, Wg, W1, W2, W3):
    b, ch, h, w = hidden_states.shape
    x = jnp.transpose(hidden_states, (0, 2, 3, 1)).reshape(-1, ch)
    ei, ew = _router(x, Wg)
    eid = ei.reshape(NA)
    wgt = ew.reshape(NA)
    we = _wescatter(eid, wgt).reshape(N_TOK, E)
    out_flat = _dense_mlp(x.astype(jnp.bfloat16), we, W1, W2, W3)
    out = out_flat.reshape(b, h, w, FFN)
    return jnp.transpose(out, (0, 3, 1, 2))
